# Initial kernel scaffold; baseline (speedup 1.0000x reference)
import jax
import jax.numpy as jnp
from jax.experimental import pallas as pl


def kernel(x, core0, core1, core2):
    raise NotImplementedError("write your pallas kernel here")


# speedup vs baseline: 7.1445x; 7.1445x over previous
import functools

import jax
import jax.numpy as jnp
from jax import lax
from jax.experimental import pallas as pl
from jax.experimental.pallas import tpu as pltpu
from jax.experimental.pallas import tpu_sc as plsc

_R1 = 16
_R2 = 16
_M0, _M1, _M2 = 100, 100, 100
_N0, _N1, _N2 = 4, 4, 4
_CP0 = 10000
_OUT_F = 64

_B = 16384 * 26
_XROWS = _B // 128
_NC = 2
_NS = 16
_NW = _NC * _NS
_RPW = 4 * _B // _NW
_CH = 4096
_NCH = _RPW // _CH
_L = 16


def _g_matmul_kernel(c1_ref, c2_ref, out_ref):
    out_ref[...] = jnp.dot(c1_ref[...], c2_ref[...],
                           preferred_element_type=jnp.float32)


def _p_matmul_kernel(a_ref, g_ref, out_ref):
    out_ref[...] = jnp.dot(a_ref[...], g_ref[...],
                           preferred_element_type=jnp.float32)


def _idx_kernel(x_ref, out_ref):
    v = x_ref[...]
    i0 = v // _CP0
    r = i0 * (4 * _CP0) + (v - i0 * _CP0)
    out_ref[0] = r
    out_ref[1] = r + _CP0
    out_ref[2] = r + 2 * _CP0
    out_ref[3] = r + 3 * _CP0


def _make_sc_lookup():
    mesh = plsc.VectorSubcoreMesh(core_axis_name="c", subcore_axis_name="s")

    @functools.partial(
        pl.kernel,
        mesh=mesh,
        compiler_params=pltpu.CompilerParams(use_tc_tiling_on_sc=False),
        out_type=jax.ShapeDtypeStruct((4 * _B, _L), jnp.float32),
        scratch_types=[
            pltpu.VMEM((_CH,), jnp.int32),
            pltpu.VMEM((_CH, _L), jnp.float32),
            pltpu.SemaphoreType.DMA,
        ],
    )
    def sc_lookup(ridx_hbm, tab_hbm, out_hbm, idx_v, buf, sem):
        wid = lax.axis_index("s") * _NC + lax.axis_index("c")
        base = wid * _RPW

        def chunk_body(k, carry):
            cbase = base + k * _CH
            pltpu.sync_copy(ridx_hbm.at[pl.ds(cbase, _CH)], idx_v)
            pltpu.async_copy(tab_hbm.at[idx_v], buf, sem).wait()
            pltpu.sync_copy(buf, out_hbm.at[pl.ds(cbase, _CH)])
            return carry

        lax.fori_loop(0, _NCH, chunk_body, 0)

    return sc_lookup


_sc_lookup = _make_sc_lookup()


def kernel(x, core0, core1, core2):
    xshape = list(x.shape)
    xf = jnp.reshape(x, (-1,)).astype(jnp.int32)

    c1m = jnp.reshape(core1, (_R1 * _M1 * _N1, _R2))
    c2m = jnp.reshape(core2, (_R2, _M2 * _N2))
    gp = pl.pallas_call(
        _g_matmul_kernel,
        out_shape=jax.ShapeDtypeStruct((_R1 * _M1 * _N1, _M2 * _N2),
                                       jnp.float32),
    )(c1m, c2m)

    g5 = jnp.reshape(
        jnp.transpose(jnp.reshape(gp, (_R1, _M1, _N1, _M2, _N2)),
                      (0, 1, 3, 2, 4)),
        (_R1, _M1 * _M2 * _N1 * _N2))

    am = jnp.reshape(core0, (_M0 * _N0, _R1))
    ncols = _M1 * _M2 * _N1 * _N2
    nblk = 50
    blk = ncols // nblk
    p = pl.pallas_call(
        _p_matmul_kernel,
        grid=(nblk,),
        in_specs=[
            pl.BlockSpec((_M0 * _N0, _R1), lambda j: (0, 0)),
            pl.BlockSpec((_R1, blk), lambda j: (0, j)),
        ],
        out_specs=pl.BlockSpec((_M0 * _N0, blk), lambda j: (0, j)),
        out_shape=jax.ShapeDtypeStruct((_M0 * _N0, ncols), jnp.float32),
    )(am, g5)

    tab = jnp.reshape(p, (_M0 * _N0 * _M1 * _M2, _L))

    x2 = jnp.reshape(xf, (_XROWS, 128))
    ridx = pl.pallas_call(
        _idx_kernel,
        out_shape=jax.ShapeDtypeStruct((4, _XROWS, 128), jnp.int32),
    )(x2)
    ridx_flat = jnp.reshape(ridx, (4 * _B,))

    rows = _sc_lookup(ridx_flat, tab)
    out = jnp.transpose(jnp.reshape(rows, (4, _B, _L)), (1, 0, 2))
    return jnp.reshape(out, tuple(xshape) + (_OUT_F,))
